# baseline (speedup 1.0000x reference)
import jax
import jax.numpy as jnp
from jax import lax
from jax.experimental import pallas as pl
from jax.experimental.pallas import tpu as pltpu

_T = 2.0
_C = 19
_R = 256
_M = _R * 512


def _body(g_ref, x_ref, out_ref, acc_s, acc_c, acc_l):
    i = pl.program_id(0)
    j = pl.program_id(1)

    @pl.when(jnp.logical_and(i == 0, j == 0))
    def _init():
        acc_s[...] = jnp.zeros_like(acc_s)
        acc_c[...] = jnp.zeros_like(acc_c)
        acc_l[0] = 0.0

    x3 = x_ref[...]
    g3 = g_ref[0]
    _, R, W = x3.shape
    M = R * W

    xs = x3 * (1.0 / _T)
    m = jnp.max(xs[0])
    e = jnp.exp(xs - m)
    s = jnp.sum(e, axis=0)
    acc_l[0] += jnp.sum(jnp.log(s)) + M * m

    g2 = g3.reshape(1, M)
    cls = lax.broadcasted_iota(jnp.int32, (_C, M), 0)
    oh2 = (g2 == cls).astype(jnp.float32)
    x2 = x3.reshape(_C, M)
    acc_s[...] += lax.dot_general(oh2, x2, (((1,), (1,)), ((), ())),
                                  preferred_element_type=jnp.float32)
    ones = jnp.ones((1, M), jnp.float32)
    acc_c[...] += lax.dot_general(oh2, ones, (((1,), (1,)), ((), ())),
                                  preferred_element_type=jnp.float32)

    @pl.when(jnp.logical_and(i == pl.num_programs(0) - 1,
                             j == pl.num_programs(1) - 1))
    def _fin():
        S = acc_s[...]
        Cn = acc_c[...]
        mean = S / jnp.maximum(Cn, 1.0)
        z = mean * (1.0 / _T)
        zm = jnp.max(z, axis=1, keepdims=True)
        ez = jnp.exp(z - zm)
        sz = jnp.sum(ez, axis=1, keepdims=True)
        q = ez / sz
        logq = (z - zm) - jnp.log(sz)
        term1 = jnp.sum(Cn * jnp.sum(q * logq, axis=1, keepdims=True))
        term2 = (1.0 / _T) * jnp.sum(q * S)
        out_ref[0] = (term1 - term2 + acc_l[0]) * (_T * _T)


def kernel(main_out, gts):
    nimg, H, W = main_out.shape[0], main_out.shape[1], main_out.shape[2]
    N = nimg * H * W
    R = _R if H % _R == 0 else H
    xt = jnp.transpose(main_out, (0, 3, 1, 2)).reshape(nimg * _C, H, W)
    g = gts.astype(jnp.int32)

    kl = pl.pallas_call(
        _body,
        grid=(nimg, H // R),
        in_specs=[
            pl.BlockSpec((1, R, W), lambda i, j: (i, j, 0)),
            pl.BlockSpec((_C, R, W), lambda i, j: (i, j, 0)),
        ],
        out_specs=pl.BlockSpec(memory_space=pltpu.SMEM),
        out_shape=jax.ShapeDtypeStruct((1,), jnp.float32),
        scratch_shapes=[
            pltpu.VMEM((_C, _C), jnp.float32),
            pltpu.VMEM((_C, 1), jnp.float32),
            pltpu.SMEM((1,), jnp.float32),
        ],
    )(g, xt)
    return kl[0] / N


# speedup vs baseline: 1.0090x; 1.0090x over previous
import jax
import jax.numpy as jnp
from jax import lax
from jax.experimental import pallas as pl
from jax.experimental.pallas import tpu as pltpu

_T = 2.0
_C = 19
_R = 128
_M = _R * 512


def _body(g_ref, x_ref, out_ref, acc_s, acc_c, acc_l):
    i = pl.program_id(0)
    j = pl.program_id(1)

    @pl.when(jnp.logical_and(i == 0, j == 0))
    def _init():
        acc_s[...] = jnp.zeros_like(acc_s)
        acc_c[...] = jnp.zeros_like(acc_c)
        acc_l[0] = 0.0

    x3 = x_ref[...]
    g3 = g_ref[0]
    _, R, W = x3.shape
    M = R * W

    xs = x3 * (1.0 / _T)
    m = jnp.max(xs[0])
    e = jnp.exp(xs - m)
    s = jnp.sum(e, axis=0)
    acc_l[0] += jnp.sum(jnp.log(s)) + M * m

    g2 = g3.reshape(1, M)
    cls = lax.broadcasted_iota(jnp.int32, (_C, M), 0)
    oh2 = (g2 == cls).astype(jnp.float32)
    x2 = x3.reshape(_C, M)
    acc_s[...] += lax.dot_general(oh2, x2, (((1,), (1,)), ((), ())),
                                  preferred_element_type=jnp.float32)
    ones = jnp.ones((1, M), jnp.float32)
    acc_c[...] += lax.dot_general(oh2, ones, (((1,), (1,)), ((), ())),
                                  preferred_element_type=jnp.float32)

    @pl.when(jnp.logical_and(i == pl.num_programs(0) - 1,
                             j == pl.num_programs(1) - 1))
    def _fin():
        S = acc_s[...]
        Cn = acc_c[...]
        mean = S / jnp.maximum(Cn, 1.0)
        z = mean * (1.0 / _T)
        zm = jnp.max(z, axis=1, keepdims=True)
        ez = jnp.exp(z - zm)
        sz = jnp.sum(ez, axis=1, keepdims=True)
        q = ez / sz
        logq = (z - zm) - jnp.log(sz)
        term1 = jnp.sum(Cn * jnp.sum(q * logq, axis=1, keepdims=True))
        term2 = (1.0 / _T) * jnp.sum(q * S)
        out_ref[0] = (term1 - term2 + acc_l[0]) * (_T * _T)


def kernel(main_out, gts):
    nimg, H, W = main_out.shape[0], main_out.shape[1], main_out.shape[2]
    N = nimg * H * W
    R = _R if H % _R == 0 else H
    xt = jnp.transpose(main_out, (0, 3, 1, 2)).reshape(nimg * _C, H, W)
    g = gts.astype(jnp.int32)

    kl = pl.pallas_call(
        _body,
        grid=(nimg, H // R),
        in_specs=[
            pl.BlockSpec((1, R, W), lambda i, j: (i, j, 0)),
            pl.BlockSpec((_C, R, W), lambda i, j: (i, j, 0)),
        ],
        out_specs=pl.BlockSpec(memory_space=pltpu.SMEM),
        out_shape=jax.ShapeDtypeStruct((1,), jnp.float32),
        scratch_shapes=[
            pltpu.VMEM((_C, _C), jnp.float32),
            pltpu.VMEM((_C, 1), jnp.float32),
            pltpu.SMEM((1,), jnp.float32),
        ],
    )(g, xt)
    return kl[0] / N
